# baseline (speedup 1.0000x reference)
import functools

import jax
import jax.numpy as jnp
from jax import lax
from jax.experimental import pallas as pl
from jax.experimental.pallas import tpu as pltpu
from jax.experimental.pallas import tpu_sc as plsc

N = 10000
NPAD = 10240
E = 320000
EPAD = 327680
D = 128
NC, NS = 2, 16
NW = NC * NS
CHUNK = 64
CPT = EPAD // NW // CHUNK
RPT = NPAD // NS

_mesh = plsc.VectorSubcoreMesh(core_axis_name="c", subcore_axis_name="s")


@functools.partial(
    pl.kernel,
    out_type=jax.ShapeDtypeStruct((NC, NPAD, D), jnp.float32),
    mesh=_mesh,
    scratch_types=[
        pltpu.VMEM((CPT, CHUNK), jnp.int32),
        pltpu.VMEM((CHUNK, D), jnp.float32),
        pltpu.VMEM_SHARED((NPAD, D), jnp.float32),
    ],
)
def _deg_kernel(dst_hbm, ones_hbm, zeros_hbm, deg_out, idx_v, ones_v, deg_sh):
    c = lax.axis_index("c")
    s = lax.axis_index("s")
    wid = s * NC + c
    pltpu.sync_copy(zeros_hbm.at[pl.ds(s * RPT, RPT)], deg_sh.at[pl.ds(s * RPT, RPT)])
    pltpu.sync_copy(ones_hbm, ones_v)
    pltpu.sync_copy(dst_hbm.at[pl.ds(wid * CPT, CPT)], idx_v)
    plsc.subcore_barrier()

    def body(j, carry):
        pltpu.sync_copy(ones_v, deg_sh.at[idx_v.at[j]], add=True)
        return carry

    lax.fori_loop(0, CPT, body, 0)
    plsc.subcore_barrier()
    pltpu.sync_copy(deg_sh.at[pl.ds(s * RPT, RPT)],
                    deg_out.at[c, pl.ds(s * RPT, RPT)])


W = 32
K0 = 64
K1 = 256


@functools.partial(
    pl.kernel,
    out_type=jax.ShapeDtypeStruct((NC, NPAD, D), jnp.float32),
    mesh=_mesh,
    scratch_types=[
        pltpu.VMEM((W, CHUNK), jnp.int32),
        pltpu.VMEM((W, CHUNK), jnp.int32),
        pltpu.VMEM((CHUNK, D), jnp.float32),
        pltpu.VMEM((CHUNK, D), jnp.float32),
        pltpu.SemaphoreType.DMA,
        pltpu.SemaphoreType.DMA,
        pltpu.VMEM_SHARED((NPAD, D), jnp.float32),
    ],
)
def _gs_kernel(tab_hbm, src_hbm, dst_hbm, zeros_hbm, agg_out,
               sidx, didx, g0, g1, sem0, sem1, acc_sh):
    c = lax.axis_index("c")
    s = lax.axis_index("s")
    pltpu.sync_copy(zeros_hbm.at[pl.ds(s * RPT, RPT)], acc_sh.at[pl.ds(s * RPT, RPT)])
    plsc.subcore_barrier()

    nwin = jnp.where(c == 0, K0 // W, K1 // W)
    base = jnp.where(c == 0, s * K0, 16 * K0 + s * K1)
    bufs = ((g0, sem0), (g1, sem1))

    def window(w, carry):
        wb0 = base + w * W
        pltpu.sync_copy(src_hbm.at[pl.ds(wb0, W)], sidx)
        pltpu.sync_copy(dst_hbm.at[pl.ds(wb0, W)], didx)
        for b in range(2):
            pltpu.async_copy(tab_hbm.at[sidx.at[b]], bufs[b][0], bufs[b][1])

        def body(k, carry2):
            for b in range(2):
                g, sem = bufs[b]
                j = 2 * k + b
                pltpu.make_async_copy(tab_hbm.at[sidx.at[j]], g, sem).wait()
                pltpu.sync_copy(g, acc_sh.at[didx.at[j]], add=True)
                pltpu.async_copy(tab_hbm.at[sidx.at[j + 2]], g, sem)
            return carry2

        lax.fori_loop(0, W // 2 - 1, body, 0)
        for b in range(2):
            g, sem = bufs[b]
            j = W - 2 + b
            pltpu.make_async_copy(tab_hbm.at[sidx.at[j]], g, sem).wait()
            pltpu.sync_copy(g, acc_sh.at[didx.at[j]], add=True)
        return carry

    lax.fori_loop(0, nwin, window, 0)
    plsc.subcore_barrier()
    pltpu.sync_copy(acc_sh.at[pl.ds(s * RPT, RPT)],
                    agg_out.at[c, pl.ds(s * RPT, RPT)])


def _scale_body(deg_ref, x_ref, xs_ref, norm_ref):
    deg = jnp.sum(deg_ref[...], axis=(0, 2)) * (1.0 / D)
    norm = lax.rsqrt(jnp.maximum(deg, 1.0))[:, None]
    xs_ref[...] = x_ref[...] * norm
    norm_ref[...] = norm


def _scale_call(deg2, x_pad):
    R = 1024
    return pl.pallas_call(
        _scale_body,
        grid=(NPAD // R,),
        in_specs=[
            pl.BlockSpec((NC, R, D), lambda i: (0, i, 0)),
            pl.BlockSpec((R, D), lambda i: (i, 0)),
        ],
        out_specs=[
            pl.BlockSpec((R, D), lambda i: (i, 0)),
            pl.BlockSpec((R, 1), lambda i: (i, 0)),
        ],
        out_shape=[
            jax.ShapeDtypeStruct((NPAD, D), jnp.float32),
            jax.ShapeDtypeStruct((NPAD, 1), jnp.float32),
        ],
    )(deg2, x_pad)


def _mm_body(relu, postscale, agg_ref, norm_ref, w_ref, b_ref, out_ref):
    norm = norm_ref[...]
    a = (agg_ref[0] + agg_ref[1]) * norm
    h = jnp.dot(a, w_ref[...], preferred_element_type=jnp.float32) + b_ref[...]
    if relu:
        h = jnp.maximum(h, 0.0)
    if postscale:
        h = h * norm
    out_ref[...] = h


def _mm_call(agg2, norm, W, b, relu, postscale):
    R = 1024
    return pl.pallas_call(
        functools.partial(_mm_body, relu, postscale),
        grid=(NPAD // R,),
        in_specs=[
            pl.BlockSpec((NC, R, D), lambda i: (0, i, 0)),
            pl.BlockSpec((R, 1), lambda i: (i, 0)),
            pl.BlockSpec((D, D), lambda i: (0, 0)),
            pl.BlockSpec((1, D), lambda i: (0, 0)),
        ],
        out_specs=pl.BlockSpec((R, D), lambda i: (i, 0)),
        out_shape=jax.ShapeDtypeStruct((NPAD, D), jnp.float32),
    )(agg2, norm, W, b.reshape(1, D))


def kernel(x, edge_index, W1, b1, W2, b2):
    src = edge_index[0].astype(jnp.int32)
    dst = edge_index[1].astype(jnp.int32)
    pad = jnp.full((EPAD - E,), N, jnp.int32)
    src2 = jnp.concatenate([src, pad]).reshape(EPAD // CHUNK, CHUNK)
    dst2 = jnp.concatenate([dst, pad]).reshape(EPAD // CHUNK, CHUNK)
    x_pad = jnp.pad(x, ((0, NPAD - N), (0, 0)))
    onesD = jnp.ones((CHUNK, D), jnp.float32)
    zerosD = jnp.zeros((NPAD, D), jnp.float32)

    deg2 = _deg_kernel(dst2, onesD, zerosD)
    xs, norm = _scale_call(deg2, x_pad)
    agg1 = _gs_kernel(xs, src2, dst2, zerosD)
    h = _mm_call(agg1, norm, W1, b1, relu=True, postscale=True)
    agg2 = _gs_kernel(h, src2, dst2, zerosD)
    out = _mm_call(agg2, norm, W2, b2, relu=False, postscale=False)
    return out[:N]


# speedup vs baseline: 1.1885x; 1.1885x over previous
import functools

import jax
import jax.numpy as jnp
from jax import lax
from jax.experimental import pallas as pl
from jax.experimental.pallas import tpu as pltpu
from jax.experimental.pallas import tpu_sc as plsc

N = 10000
NPAD = 10240
E = 320000
EPAD = 327680
D = 128
NC, NS = 2, 16
NW = NC * NS
CHUNK = 64
CPT = EPAD // NW // CHUNK
RPT = NPAD // NS

_mesh = plsc.VectorSubcoreMesh(core_axis_name="c", subcore_axis_name="s")


@functools.partial(
    pl.kernel,
    out_type=jax.ShapeDtypeStruct((NC, NPAD, D), jnp.float32),
    mesh=_mesh,
    scratch_types=[
        pltpu.VMEM((CPT, CHUNK), jnp.int32),
        pltpu.VMEM((CHUNK, D), jnp.float32),
        pltpu.VMEM_SHARED((NPAD, D), jnp.float32),
    ],
)
def _deg_kernel(dst_hbm, ones_hbm, zeros_hbm, deg_out, idx_v, ones_v, deg_sh):
    c = lax.axis_index("c")
    s = lax.axis_index("s")
    wid = s * NC + c
    pltpu.sync_copy(zeros_hbm.at[pl.ds(s * RPT, RPT)], deg_sh.at[pl.ds(s * RPT, RPT)])
    pltpu.sync_copy(ones_hbm, ones_v)
    pltpu.sync_copy(dst_hbm.at[pl.ds(wid * CPT, CPT)], idx_v)
    plsc.subcore_barrier()

    def body(j, carry):
        pltpu.sync_copy(ones_v, deg_sh.at[idx_v.at[j]], add=True)
        return carry

    lax.fori_loop(0, CPT, body, 0)
    plsc.subcore_barrier()
    pltpu.sync_copy(deg_sh.at[pl.ds(s * RPT, RPT)],
                    deg_out.at[c, pl.ds(s * RPT, RPT)])


W = 32
K0 = 256
K1 = 64


@functools.partial(
    pl.kernel,
    out_type=jax.ShapeDtypeStruct((NC, NPAD, D), jnp.float32),
    mesh=_mesh,
    scratch_types=[
        pltpu.VMEM((W, CHUNK), jnp.int32),
        pltpu.VMEM((W, CHUNK), jnp.int32),
        pltpu.VMEM((CHUNK, D), jnp.float32),
        pltpu.VMEM((CHUNK, D), jnp.float32),
        pltpu.SemaphoreType.DMA,
        pltpu.SemaphoreType.DMA,
        pltpu.VMEM_SHARED((NPAD, D), jnp.float32),
    ],
)
def _gs_kernel(tab_hbm, src_hbm, dst_hbm, zeros_hbm, agg_out,
               sidx, didx, g0, g1, sem0, sem1, acc_sh):
    c = lax.axis_index("c")
    s = lax.axis_index("s")
    pltpu.sync_copy(zeros_hbm.at[pl.ds(s * RPT, RPT)], acc_sh.at[pl.ds(s * RPT, RPT)])
    plsc.subcore_barrier()

    nwin = jnp.where(c == 0, K0 // W, K1 // W)
    base = jnp.where(c == 0, s * K0, 16 * K0 + s * K1)
    bufs = ((g0, sem0), (g1, sem1))

    def window(w, carry):
        wb0 = base + w * W
        pltpu.sync_copy(src_hbm.at[pl.ds(wb0, W)], sidx)
        pltpu.sync_copy(dst_hbm.at[pl.ds(wb0, W)], didx)
        for b in range(2):
            pltpu.async_copy(tab_hbm.at[sidx.at[b]], bufs[b][0], bufs[b][1])

        def body(k, carry2):
            for b in range(2):
                g, sem = bufs[b]
                j = 2 * k + b
                pltpu.make_async_copy(tab_hbm.at[sidx.at[j]], g, sem).wait()
                pltpu.sync_copy(g, acc_sh.at[didx.at[j]], add=True)
                pltpu.async_copy(tab_hbm.at[sidx.at[j + 2]], g, sem)
            return carry2

        lax.fori_loop(0, W // 2 - 1, body, 0)
        for b in range(2):
            g, sem = bufs[b]
            j = W - 2 + b
            pltpu.make_async_copy(tab_hbm.at[sidx.at[j]], g, sem).wait()
            pltpu.sync_copy(g, acc_sh.at[didx.at[j]], add=True)
        return carry

    lax.fori_loop(0, nwin, window, 0)
    plsc.subcore_barrier()
    pltpu.sync_copy(acc_sh.at[pl.ds(s * RPT, RPT)],
                    agg_out.at[c, pl.ds(s * RPT, RPT)])


def _scale_body(deg_ref, x_ref, xs_ref, norm_ref):
    deg = jnp.sum(deg_ref[...], axis=(0, 2)) * (1.0 / D)
    norm = lax.rsqrt(jnp.maximum(deg, 1.0))[:, None]
    xs_ref[...] = x_ref[...] * norm
    norm_ref[...] = norm


def _scale_call(deg2, x_pad):
    R = 1024
    return pl.pallas_call(
        _scale_body,
        grid=(NPAD // R,),
        in_specs=[
            pl.BlockSpec((NC, R, D), lambda i: (0, i, 0)),
            pl.BlockSpec((R, D), lambda i: (i, 0)),
        ],
        out_specs=[
            pl.BlockSpec((R, D), lambda i: (i, 0)),
            pl.BlockSpec((R, 1), lambda i: (i, 0)),
        ],
        out_shape=[
            jax.ShapeDtypeStruct((NPAD, D), jnp.float32),
            jax.ShapeDtypeStruct((NPAD, 1), jnp.float32),
        ],
    )(deg2, x_pad)


def _mm_body(relu, postscale, agg_ref, norm_ref, w_ref, b_ref, out_ref):
    norm = norm_ref[...]
    a = (agg_ref[0] + agg_ref[1]) * norm
    h = jnp.dot(a, w_ref[...], preferred_element_type=jnp.float32) + b_ref[...]
    if relu:
        h = jnp.maximum(h, 0.0)
    if postscale:
        h = h * norm
    out_ref[...] = h


def _mm_call(agg2, norm, W, b, relu, postscale):
    R = 1024
    return pl.pallas_call(
        functools.partial(_mm_body, relu, postscale),
        grid=(NPAD // R,),
        in_specs=[
            pl.BlockSpec((NC, R, D), lambda i: (0, i, 0)),
            pl.BlockSpec((R, 1), lambda i: (i, 0)),
            pl.BlockSpec((D, D), lambda i: (0, 0)),
            pl.BlockSpec((1, D), lambda i: (0, 0)),
        ],
        out_specs=pl.BlockSpec((R, D), lambda i: (i, 0)),
        out_shape=jax.ShapeDtypeStruct((NPAD, D), jnp.float32),
    )(agg2, norm, W, b.reshape(1, D))


def kernel(x, edge_index, W1, b1, W2, b2):
    src = edge_index[0].astype(jnp.int32)
    dst = edge_index[1].astype(jnp.int32)
    pad = jnp.full((EPAD - E,), N, jnp.int32)
    src2 = jnp.concatenate([src, pad]).reshape(EPAD // CHUNK, CHUNK)
    dst2 = jnp.concatenate([dst, pad]).reshape(EPAD // CHUNK, CHUNK)
    x_pad = jnp.pad(x, ((0, NPAD - N), (0, 0)))
    onesD = jnp.ones((CHUNK, D), jnp.float32)
    zerosD = jnp.zeros((NPAD, D), jnp.float32)

    deg2 = _deg_kernel(dst2, onesD, zerosD)
    xs, norm = _scale_call(deg2, x_pad)
    agg1 = _gs_kernel(xs, src2, dst2, zerosD)
    h = _mm_call(agg1, norm, W1, b1, relu=True, postscale=True)
    agg2 = _gs_kernel(h, src2, dst2, zerosD)
    out = _mm_call(agg2, norm, W2, b2, relu=False, postscale=False)
    return out[:N]
